# baseline (speedup 1.0000x reference)
import jax
import jax.numpy as jnp
from jax import lax
from jax.experimental import pallas as pl
from jax.experimental.pallas import tpu as pltpu
from jax.experimental.pallas import tpu_sc as plsc

NUM_GRAPHS = 1024
HIDDEN = 14
N_NODES = 100000

NC = 2
NS = 16
NW = NC * NS
CHUNK = 3200
GB = 128
G = CHUNK // GB
N_PAD = NW * CHUNK
ROWS_PER_TILE = NUM_GRAPHS // NS
LAST_REAL = N_NODES - (NW - 1) * CHUNK
LAST_FULL = LAST_REAL // GB
LAST_TAIL = LAST_REAL - LAST_FULL * GB


def _seg_body(x_hbm, idx_hbm, z_hbm, zc_hbm, ones_hbm, out_s, out_c,
              xv0, xv1, idxv, onesv, accs, accc, sem):
    cid = lax.axis_index("c")
    sid = lax.axis_index("s")
    wid = cid * NS + sid
    base = wid * CHUNK

    pltpu.sync_copy(z_hbm, accs.at[pl.ds(sid * ROWS_PER_TILE, ROWS_PER_TILE), :])
    pltpu.sync_copy(zc_hbm, accc.at[pl.ds(sid * ROWS_PER_TILE, ROWS_PER_TILE), :])
    pltpu.sync_copy(ones_hbm, onesv)
    for g in range(G):
        pltpu.sync_copy(idx_hbm.at[pl.ds(base + g * GB, GB)], idxv.at[g, :])

    plsc.subcore_barrier()

    def scatter(buf, g):
        pltpu.sync_copy(buf, accs.at[idxv.at[g, :]], add=True)
        pltpu.sync_copy(onesv, accc.at[idxv.at[g, :]], add=True)

    @pl.when(wid < NW - 1)
    def _():
        h = pltpu.async_copy(x_hbm.at[pl.ds(base, GB), :], xv0, sem)
        for g in range(G):
            buf, nbuf = (xv0, xv1) if g % 2 == 0 else (xv1, xv0)
            h.wait()
            if g + 1 < G:
                h = pltpu.async_copy(
                    x_hbm.at[pl.ds(base + (g + 1) * GB, GB), :], nbuf, sem)
            scatter(buf, g)

    @pl.when(wid == NW - 1)
    def _():
        h = pltpu.async_copy(x_hbm.at[pl.ds(base, GB), :], xv0, sem)
        for g in range(LAST_FULL):
            buf, nbuf = (xv0, xv1) if g % 2 == 0 else (xv1, xv0)
            h.wait()
            if g + 1 < LAST_FULL:
                h = pltpu.async_copy(
                    x_hbm.at[pl.ds(base + (g + 1) * GB, GB), :], nbuf, sem)
            scatter(buf, g)
        tbuf = xv0 if LAST_FULL % 2 == 0 else xv1
        pltpu.sync_copy(x_hbm.at[pl.ds(base + LAST_FULL * GB, LAST_TAIL), :],
                        tbuf.at[pl.ds(0, LAST_TAIL), :])
        scatter(tbuf, LAST_FULL)

    plsc.subcore_barrier()

    pltpu.sync_copy(accs.at[pl.ds(sid * ROWS_PER_TILE, ROWS_PER_TILE), :],
                    out_s.at[cid, pl.ds(sid * ROWS_PER_TILE, ROWS_PER_TILE), :])
    pltpu.sync_copy(accc.at[pl.ds(sid * ROWS_PER_TILE, ROWS_PER_TILE), :],
                    out_c.at[cid, pl.ds(sid * ROWS_PER_TILE, ROWS_PER_TILE), :])


_seg_kernel = pl.kernel(
    _seg_body,
    out_type=(
        jax.ShapeDtypeStruct((NC, NUM_GRAPHS, HIDDEN), jnp.float32),
        jax.ShapeDtypeStruct((NC, NUM_GRAPHS, 1), jnp.float32),
    ),
    mesh=plsc.VectorSubcoreMesh(core_axis_name="c", subcore_axis_name="s",
                                num_cores=NC, num_subcores=NS),
    scratch_types=[
        pltpu.VMEM((GB, HIDDEN), jnp.float32),
        pltpu.VMEM((GB, HIDDEN), jnp.float32),
        pltpu.VMEM((G, GB), jnp.int32),
        pltpu.VMEM((GB, 1), jnp.float32),
        pltpu.VMEM_SHARED((NUM_GRAPHS + 1, HIDDEN), jnp.float32),
        pltpu.VMEM_SHARED((NUM_GRAPHS + 1, 1), jnp.float32),
        pltpu.SemaphoreType.DMA,
    ],
    compiler_params=pltpu.CompilerParams(use_tc_tiling_on_sc=True),
)


def _mlp_body(ps, pc, w1t, b1, w2t, b2, o):
    cnt = jnp.maximum(pc[0] + pc[1], 1.0)
    mean = (ps[0] + ps[1]) / cnt
    h = jnp.maximum(
        jnp.dot(mean, w1t[...], preferred_element_type=jnp.float32) + b1[...], 0.0)
    o[...] = jnp.dot(h, w2t[...], preferred_element_type=jnp.float32) + b2[...]


def _mlp(ps, pc, w1t, b1, w2t, b2):
    return pl.pallas_call(
        _mlp_body,
        out_shape=jax.ShapeDtypeStruct((NUM_GRAPHS, 2), jnp.float32),
    )(ps, pc, w1t, b1, w2t, b2)


def kernel(x, edge_index, edge_attr, u, batch, W1, b1, W2, b2):
    idx = jnp.pad(batch.astype(jnp.int32), (0, N_PAD - N_NODES),
                  constant_values=NUM_GRAPHS)
    z = jnp.zeros((ROWS_PER_TILE, HIDDEN), jnp.float32)
    zc = jnp.zeros((ROWS_PER_TILE, 1), jnp.float32)
    ones = jnp.ones((GB, 1), jnp.float32)
    ps, pc = _seg_kernel(x, idx, z, zc, ones)
    return _mlp(ps, pc, W1.T, b1[None, :], W2.T, b2[None, :])
